# baseline (speedup 1.0000x reference)
import functools

import jax
import jax.numpy as jnp
from jax import lax
from jax.experimental import pallas as pl
from jax.experimental.pallas import tpu as pltpu
from jax.experimental.pallas import tpu_sc as plsc

NEG = -1e38
IMAX = 2**31 - 1


def _sc_row_gather(table, idx):
    b = idx.shape[0]
    v, d = table.shape
    n = b * d
    try:
        info = plsc.get_sparse_core_info()
        nw = info.num_cores * info.num_subcores
    except Exception:
        nw = 0
    if nw == 0 or n % (8 * nw) != 0:
        return jnp.take(table, idx, axis=0)
    n_per_w = n // nw
    flat_idx = (idx[:, None] * d
                + jnp.arange(d, dtype=jnp.int32)[None, :]).reshape(n)
    flat_tab = table.reshape(n if v * d == n else v * d)
    mesh = plsc.VectorSubcoreMesh(core_axis_name="c", subcore_axis_name="s")

    @functools.partial(
        pl.kernel, mesh=mesh,
        out_type=jax.ShapeDtypeStruct((n,), jnp.float32),
        scratch_types=[
            pltpu.VMEM((n_per_w,), jnp.int32),
            pltpu.VMEM((n_per_w,), jnp.float32),
            pltpu.SemaphoreType.DMA,
        ],
    )
    def gather_kernel(table_hbm, idx_hbm, out_hbm, idx_v, rows_v, sem):
        wid = lax.axis_index("s") * info.num_cores + lax.axis_index("c")
        base = wid * n_per_w
        pltpu.sync_copy(idx_hbm.at[pl.ds(base, n_per_w)], idx_v)
        pltpu.async_copy(table_hbm.at[idx_v], rows_v, sem).wait()
        pltpu.sync_copy(rows_v, out_hbm.at[pl.ds(base, n_per_w)])

    return gather_kernel(flat_tab, flat_idx).reshape(b, d)


def _topk_body(nb, qt, bk, q_ref, kt_ref, psi_ref, vals_out, idx_out,
               s_ref, cm_ref, t_ref, rv_ref, ri_ref):
    j = pl.program_id(1)
    jb = jnp.where(j < nb, j, j - nb)
    nchunk = bk // 128

    @pl.when(j == 0)
    def _init():
        rv_ref[:, :] = jnp.full((qt, 16), NEG, jnp.float32)
        ri_ref[:, :] = jnp.full((qt, 16), IMAX, jnp.int32)

    @pl.when(j < nb)
    def _sweep_a():
        s = jnp.dot(q_ref[:, :], kt_ref[:, :],
                    preferred_element_type=jnp.float32) - psi_ref[:, :]
        colmax = s[:, 0:128]
        for t in range(1, nchunk):
            colmax = jnp.maximum(colmax, s[:, t * 128:(t + 1) * 128])
        cm_ref[jb] = colmax

    @pl.when(j == nb)
    def _threshold():
        ngrp = -(-nb // 8)
        folds = []
        for g in range(ngrp):
            f = cm_ref[8 * g]
            for b in range(8 * g + 1, min(8 * g + 8, nb)):
                f = jnp.maximum(f, cm_ref[b])
            folds.append(f)
        m = None
        for s16 in range(16):
            red = folds[0]
            for f in folds[1:]:
                red = jnp.maximum(red, f)
            m = jnp.max(red, axis=1, keepdims=True)
            if s16 < 15:
                folds = [jnp.where(f == m, NEG, f) for f in folds]
        t_ref[:, :] = m

    @pl.when(j >= nb)
    def _sweep_b():
        s_ref[:, :] = jnp.dot(q_ref[:, :], kt_ref[:, :],
                              preferred_element_type=jnp.float32) - psi_ref[:, :]
        liota = jax.lax.broadcasted_iota(jnp.int32, (qt, bk), 1)
        lane16 = jax.lax.broadcasted_iota(jnp.int32, (qt, 16), 1)
        tfloor = t_ref[:, :]

        def body(carry):
            t, _, m = carry
            s = s_ref[:, :]
            rv = rv_ref[:, :]
            ri = ri_ref[:, :]
            upd = (m >= tfloor) & (m > rv[:, 15:16])
            sel = jnp.min(jnp.where(s == m, liota, IMAX), axis=1,
                          keepdims=True)
            g = sel + jb * bk
            above = (rv > m) | ((rv == m) & (ri < g))
            pos = jnp.sum(above.astype(jnp.int32), axis=1, keepdims=True)
            rolled_v = jnp.roll(rv, 1, axis=1)
            rolled_i = jnp.roll(ri, 1, axis=1)
            nrv = jnp.where(lane16 < pos, rv,
                            jnp.where(lane16 == pos, m, rolled_v))
            nri = jnp.where(lane16 < pos, ri,
                            jnp.where(lane16 == pos, g, rolled_i))
            nrv = jnp.where(upd, nrv, rv)
            nri = jnp.where(upd, nri, ri)
            rv_ref[:, :] = nrv
            ri_ref[:, :] = nri
            s2 = jnp.where(liota == sel, NEG, s)
            s_ref[:, :] = s2
            m2 = jnp.max(s2, axis=1, keepdims=True)
            cont = jnp.any((m2 >= tfloor) & (m2 > nrv[:, 15:16]))
            return t + jnp.int32(1), cont, m2

        def cond(carry):
            t, cont, _ = carry
            return jnp.logical_and(t < 16, cont)

        m0 = jnp.max(cm_ref[jb], axis=1, keepdims=True)
        cont0 = jnp.any((m0 >= tfloor) & (m0 > rv_ref[:, 15:16]))
        jax.lax.while_loop(cond, body, (jnp.int32(0), cont0, m0))

    @pl.when(j == 2 * nb - 1)
    def _out():
        vals_out[:, :] = rv_ref[:, :]
        idx_out[:, :] = ri_ref[:, :]


def kernel(queries, keys, psi, k):
    q, d = queries.shape
    kn = keys.shape[0]
    bk = 2048
    qt = 128 if q % 128 == 0 else q
    nb = -(-kn // bk)
    kp = nb * bk

    keys_t = jnp.transpose(keys)
    if kp != kn:
        keys_t = jnp.pad(keys_t, ((0, 0), (0, kp - kn)))
        psi_p = jnp.pad(psi, (0, kp - kn), constant_values=1e30)
    else:
        psi_p = psi
    psi_p = psi_p[None, :]

    vals, idx = pl.pallas_call(
        functools.partial(_topk_body, nb, qt, bk),
        grid=(q // qt, 2 * nb),
        in_specs=[
            pl.BlockSpec((qt, d), lambda i, j: (i, 0)),
            pl.BlockSpec((d, bk), lambda i, j: (0, jnp.where(j < nb, j, j - nb))),
            pl.BlockSpec((1, bk), lambda i, j: (0, jnp.where(j < nb, j, j - nb))),
        ],
        out_specs=[
            pl.BlockSpec((qt, 16), lambda i, j: (i, 0)),
            pl.BlockSpec((qt, 16), lambda i, j: (i, 0)),
        ],
        out_shape=[
            jax.ShapeDtypeStruct((q, 16), jnp.float32),
            jax.ShapeDtypeStruct((q, 16), jnp.int32),
        ],
        scratch_shapes=[
            pltpu.VMEM((qt, bk), jnp.float32),
            pltpu.VMEM((nb, qt, 128), jnp.float32),
            pltpu.VMEM((qt, 1), jnp.float32),
            pltpu.VMEM((qt, 16), jnp.float32),
            pltpu.VMEM((qt, 16), jnp.int32),
        ],
        compiler_params=pltpu.CompilerParams(
            dimension_semantics=("parallel", "arbitrary"),
        ),
    )(queries, keys_t, psi_p)

    mapped = _sc_row_gather(keys, idx[:, 0])
    return vals, idx, mapped


# speedup vs baseline: 1.0161x; 1.0161x over previous
import functools

import jax
import jax.numpy as jnp
from jax import lax
from jax.experimental import pallas as pl
from jax.experimental.pallas import tpu as pltpu
from jax.experimental.pallas import tpu_sc as plsc

NEG = -1e38
IMAX = 2**31 - 1


def _sc_row_gather(table, idx):
    b = idx.shape[0]
    v, d = table.shape
    n = b * d
    try:
        info = plsc.get_sparse_core_info()
        nw = info.num_cores * info.num_subcores
    except Exception:
        nw = 0
    if nw == 0 or n % (8 * nw) != 0:
        return jnp.take(table, idx, axis=0)
    n_per_w = n // nw
    flat_idx = (idx[:, None] * d
                + jnp.arange(d, dtype=jnp.int32)[None, :]).reshape(n)
    flat_tab = table.reshape(n if v * d == n else v * d)
    mesh = plsc.VectorSubcoreMesh(core_axis_name="c", subcore_axis_name="s")

    @functools.partial(
        pl.kernel, mesh=mesh,
        out_type=jax.ShapeDtypeStruct((n,), jnp.float32),
        scratch_types=[
            pltpu.VMEM((n_per_w,), jnp.int32),
            pltpu.VMEM((n_per_w,), jnp.float32),
            pltpu.SemaphoreType.DMA,
        ],
    )
    def gather_kernel(table_hbm, idx_hbm, out_hbm, idx_v, rows_v, sem):
        wid = lax.axis_index("s") * info.num_cores + lax.axis_index("c")
        base = wid * n_per_w
        pltpu.sync_copy(idx_hbm.at[pl.ds(base, n_per_w)], idx_v)
        pltpu.async_copy(table_hbm.at[idx_v], rows_v, sem).wait()
        pltpu.sync_copy(rows_v, out_hbm.at[pl.ds(base, n_per_w)])

    return gather_kernel(flat_tab, flat_idx).reshape(b, d)


def _topk_body(nb, qt, bk, q_ref, kt_ref, psi_ref, vals_out, idx_out,
               s_ref, cm_ref, t_ref, rv_ref, ri_ref):
    j = pl.program_id(1)
    jb = jnp.where(j < nb, j, j - nb)
    nchunk = bk // 128

    @pl.when(j == 0)
    def _init():
        rv_ref[:, :] = jnp.full((qt, 16), NEG, jnp.float32)
        ri_ref[:, :] = jnp.full((qt, 16), IMAX, jnp.int32)

    @pl.when(j < nb)
    def _sweep_a():
        s = jnp.dot(q_ref[:, :], kt_ref[:, :],
                    preferred_element_type=jnp.float32) - psi_ref[:, :]
        s_ref[jb] = s
        colmax = s[:, 0:128]
        for t in range(1, nchunk):
            colmax = jnp.maximum(colmax, s[:, t * 128:(t + 1) * 128])
        cm_ref[jb] = colmax

    @pl.when(j == nb)
    def _threshold():
        ngrp = -(-nb // 8)
        folds = []
        for g in range(ngrp):
            f = cm_ref[8 * g]
            for b in range(8 * g + 1, min(8 * g + 8, nb)):
                f = jnp.maximum(f, cm_ref[b])
            folds.append(f)
        m = None
        for s16 in range(16):
            red = folds[0]
            for f in folds[1:]:
                red = jnp.maximum(red, f)
            m = jnp.max(red, axis=1, keepdims=True)
            if s16 < 15:
                folds = [jnp.where(f == m, NEG, f) for f in folds]
        t_ref[:, :] = m

    @pl.when(j >= nb)
    def _sweep_b():
        liota = jax.lax.broadcasted_iota(jnp.int32, (qt, bk), 1)
        lane16 = jax.lax.broadcasted_iota(jnp.int32, (qt, 16), 1)
        tfloor = t_ref[:, :]

        def body(carry):
            t, _, m = carry
            s = s_ref[jb]
            rv = rv_ref[:, :]
            ri = ri_ref[:, :]
            upd = (m >= tfloor) & (m > rv[:, 15:16])
            sel = jnp.min(jnp.where(s == m, liota, IMAX), axis=1,
                          keepdims=True)
            g = sel + jb * bk
            above = (rv > m) | ((rv == m) & (ri < g))
            pos = jnp.sum(above.astype(jnp.int32), axis=1, keepdims=True)
            rolled_v = jnp.roll(rv, 1, axis=1)
            rolled_i = jnp.roll(ri, 1, axis=1)
            nrv = jnp.where(lane16 < pos, rv,
                            jnp.where(lane16 == pos, m, rolled_v))
            nri = jnp.where(lane16 < pos, ri,
                            jnp.where(lane16 == pos, g, rolled_i))
            nrv = jnp.where(upd, nrv, rv)
            nri = jnp.where(upd, nri, ri)
            rv_ref[:, :] = nrv
            ri_ref[:, :] = nri
            s2 = jnp.where(liota == sel, NEG, s)
            s_ref[jb] = s2
            m2 = jnp.max(s2, axis=1, keepdims=True)
            cont = jnp.any((m2 >= tfloor) & (m2 > nrv[:, 15:16]))
            return t + jnp.int32(1), cont, m2

        def cond(carry):
            t, cont, _ = carry
            return jnp.logical_and(t < 16, cont)

        m0 = jnp.max(cm_ref[jb], axis=1, keepdims=True)
        cont0 = jnp.any((m0 >= tfloor) & (m0 > rv_ref[:, 15:16]))
        jax.lax.while_loop(cond, body, (jnp.int32(0), cont0, m0))

    @pl.when(j == 2 * nb - 1)
    def _out():
        vals_out[:, :] = rv_ref[:, :]
        idx_out[:, :] = ri_ref[:, :]


def kernel(queries, keys, psi, k):
    q, d = queries.shape
    kn = keys.shape[0]
    bk = 2048
    qt = 128 if q % 128 == 0 else q
    nb = -(-kn // bk)
    kp = nb * bk

    keys_t = jnp.transpose(keys)
    if kp != kn:
        keys_t = jnp.pad(keys_t, ((0, 0), (0, kp - kn)))
        psi_p = jnp.pad(psi, (0, kp - kn), constant_values=1e30)
    else:
        psi_p = psi
    psi_p = psi_p[None, :]

    vals, idx = pl.pallas_call(
        functools.partial(_topk_body, nb, qt, bk),
        grid=(q // qt, 2 * nb),
        in_specs=[
            pl.BlockSpec((qt, d), lambda i, j: (i, 0)),
            pl.BlockSpec((d, bk), lambda i, j: (0, jnp.where(j < nb, j, 0))),
            pl.BlockSpec((1, bk), lambda i, j: (0, jnp.where(j < nb, j, 0))),
        ],
        out_specs=[
            pl.BlockSpec((qt, 16), lambda i, j: (i, 0)),
            pl.BlockSpec((qt, 16), lambda i, j: (i, 0)),
        ],
        out_shape=[
            jax.ShapeDtypeStruct((q, 16), jnp.float32),
            jax.ShapeDtypeStruct((q, 16), jnp.int32),
        ],
        scratch_shapes=[
            pltpu.VMEM((nb, qt, bk), jnp.float32),
            pltpu.VMEM((nb, qt, 128), jnp.float32),
            pltpu.VMEM((qt, 1), jnp.float32),
            pltpu.VMEM((qt, 16), jnp.float32),
            pltpu.VMEM((qt, 16), jnp.int32),
        ],
        compiler_params=pltpu.CompilerParams(
            dimension_semantics=("parallel", "arbitrary"),
        ),
    )(queries, keys_t, psi_p)

    mapped = _sc_row_gather(keys, idx[:, 0])
    return vals, idx, mapped


# speedup vs baseline: 1.0162x; 1.0001x over previous
import functools

import jax
import jax.numpy as jnp
from jax import lax
from jax.experimental import pallas as pl
from jax.experimental.pallas import tpu as pltpu
from jax.experimental.pallas import tpu_sc as plsc

NEG = -1e38
IMAX = 2**31 - 1


def _sc_row_gather(table, idx):
    b = idx.shape[0]
    v, d = table.shape
    n = b * d
    try:
        info = plsc.get_sparse_core_info()
        nw = info.num_cores * info.num_subcores
    except Exception:
        nw = 0
    if nw == 0 or n % (8 * nw) != 0:
        return jnp.take(table, idx, axis=0)
    n_per_w = n // nw
    flat_idx = (idx[:, None] * d
                + jnp.arange(d, dtype=jnp.int32)[None, :]).reshape(n)
    flat_tab = table.reshape(v * d)
    mesh = plsc.VectorSubcoreMesh(core_axis_name="c", subcore_axis_name="s")

    @functools.partial(
        pl.kernel, mesh=mesh,
        out_type=jax.ShapeDtypeStruct((n,), jnp.float32),
        scratch_types=[
            pltpu.VMEM((n_per_w,), jnp.int32),
            pltpu.VMEM((n_per_w,), jnp.float32),
            pltpu.SemaphoreType.DMA,
        ],
    )
    def gather_kernel(table_hbm, idx_hbm, out_hbm, idx_v, rows_v, sem):
        wid = lax.axis_index("s") * info.num_cores + lax.axis_index("c")
        base = wid * n_per_w
        pltpu.sync_copy(idx_hbm.at[pl.ds(base, n_per_w)], idx_v)
        pltpu.async_copy(table_hbm.at[idx_v], rows_v, sem).wait()
        pltpu.sync_copy(rows_v, out_hbm.at[pl.ds(base, n_per_w)])

    return gather_kernel(flat_tab, flat_idx).reshape(b, d)


def _topk_body(nb, qt, bk, q_ref, kt_ref, psi_ref, vals_out, idx_out,
               s_ref, cm_ref, t_ref, rv_ref, ri_ref):
    j = pl.program_id(1)
    jb = jnp.where(j < nb, j, j - nb)
    nchunk = bk // 128

    @pl.when(j == 0)
    def _init():
        rv_ref[:, :] = jnp.full((qt, 16), NEG, jnp.float32)
        ri_ref[:, :] = jnp.full((qt, 16), IMAX, jnp.int32)

    @pl.when(j < nb)
    def _sweep_a():
        s = jnp.dot(q_ref[:, :], kt_ref[:, :],
                    preferred_element_type=jnp.float32) - psi_ref[:, :]
        s_ref[jb] = s
        colmax = s[:, 0:128]
        for t in range(1, nchunk):
            colmax = jnp.maximum(colmax, s[:, t * 128:(t + 1) * 128])
        cm_ref[jb] = colmax

    @pl.when(j == nb)
    def _threshold():
        ngrp = -(-nb // 8)
        folds = []
        for g in range(ngrp):
            f = cm_ref[8 * g]
            for b in range(8 * g + 1, min(8 * g + 8, nb)):
                f = jnp.maximum(f, cm_ref[b])
            folds.append(f)
        m = None
        for s16 in range(16):
            red = folds[0]
            for f in folds[1:]:
                red = jnp.maximum(red, f)
            m = jnp.max(red, axis=1, keepdims=True)
            if s16 < 15:
                folds = [jnp.where(f == m, NEG, f) for f in folds]
        t_ref[:, :] = m

    @pl.when(j >= nb)
    def _sweep_b():
        liota = jax.lax.broadcasted_iota(jnp.int32, (qt, bk), 1)
        lane16 = jax.lax.broadcasted_iota(jnp.int32, (qt, 16), 1)
        tfloor = t_ref[:, :]

        def body(carry):
            t, _, m = carry
            s = s_ref[jb]
            rv = rv_ref[:, :]
            ri = ri_ref[:, :]
            upd = (m >= tfloor) & (m > rv[:, 15:16])
            sel = jnp.min(jnp.where(s == m, liota, IMAX), axis=1,
                          keepdims=True)
            g = sel + jb * bk
            above = (rv > m) | ((rv == m) & (ri < g))
            pos = jnp.sum(above.astype(jnp.int32), axis=1, keepdims=True)
            rolled_v = jnp.roll(rv, 1, axis=1)
            rolled_i = jnp.roll(ri, 1, axis=1)
            nrv = jnp.where(lane16 < pos, rv,
                            jnp.where(lane16 == pos, m, rolled_v))
            nri = jnp.where(lane16 < pos, ri,
                            jnp.where(lane16 == pos, g, rolled_i))
            nrv = jnp.where(upd, nrv, rv)
            nri = jnp.where(upd, nri, ri)
            rv_ref[:, :] = nrv
            ri_ref[:, :] = nri
            s2 = jnp.where(liota == sel, NEG, s)
            s_ref[jb] = s2
            m2 = jnp.max(s2, axis=1, keepdims=True)
            cont = jnp.any((m2 >= tfloor) & (m2 > nrv[:, 15:16]))
            return t + jnp.int32(1), cont, m2

        def cond(carry):
            t, cont, _ = carry
            return jnp.logical_and(t < 16, cont)

        m0 = jnp.max(cm_ref[jb], axis=1, keepdims=True)
        cont0 = jnp.any((m0 >= tfloor) & (m0 > rv_ref[:, 15:16]))
        jax.lax.while_loop(cond, body, (jnp.int32(0), cont0, m0))

    @pl.when(j == 2 * nb - 1)
    def _out():
        vals_out[:, :] = rv_ref[:, :]
        idx_out[:, :] = ri_ref[:, :]


def kernel(queries, keys, psi, k):
    q, d = queries.shape
    kn = keys.shape[0]
    bk = 2048
    qt = 128 if q % 128 == 0 else q
    nb = -(-kn // bk)
    kp = nb * bk

    keys_t = jnp.transpose(keys)
    if kp != kn:
        keys_t = jnp.pad(keys_t, ((0, 0), (0, kp - kn)))
        psi_p = jnp.pad(psi, (0, kp - kn), constant_values=1e30)
    else:
        psi_p = psi
    psi_p = psi_p[None, :]

    vals, idx = pl.pallas_call(
        functools.partial(_topk_body, nb, qt, bk),
        grid=(q // qt, 2 * nb),
        in_specs=[
            pl.BlockSpec((qt, d), lambda i, j: (i, 0)),
            pl.BlockSpec((d, bk), lambda i, j: (0, jnp.where(j < nb, j, 0))),
            pl.BlockSpec((1, bk), lambda i, j: (0, jnp.where(j < nb, j, 0))),
        ],
        out_specs=[
            pl.BlockSpec((qt, 16), lambda i, j: (i, 0)),
            pl.BlockSpec((qt, 16), lambda i, j: (i, 0)),
        ],
        out_shape=[
            jax.ShapeDtypeStruct((q, 16), jnp.float32),
            jax.ShapeDtypeStruct((q, 16), jnp.int32),
        ],
        scratch_shapes=[
            pltpu.VMEM((nb, qt, bk), jnp.float32),
            pltpu.VMEM((nb, qt, 128), jnp.float32),
            pltpu.VMEM((qt, 1), jnp.float32),
            pltpu.VMEM((qt, 16), jnp.float32),
            pltpu.VMEM((qt, 16), jnp.int32),
        ],
        compiler_params=pltpu.CompilerParams(
            dimension_semantics=("parallel", "arbitrary"),
        ),
    )(queries, keys_t, psi_p)

    mapped = _sc_row_gather(keys, idx[:, 0])
    return vals, idx, mapped
